# baseline (speedup 1.0000x reference)
import functools

import jax
import jax.numpy as jnp
from jax import lax
from jax.experimental import pallas as pl
from jax.experimental.pallas import tpu as pltpu
from jax.experimental.pallas import tpu_sc as plsc


def _tc_add_body(x_ref, emb_ref, o_ref):
    o_ref[...] = x_ref[...] + emb_ref[...]


def _tc_add(x, emb_s, nb_tc):
    B, S, D = x.shape
    S_BLK = 1024
    return pl.pallas_call(
        _tc_add_body,
        grid=(S // S_BLK,),
        in_specs=[
            pl.BlockSpec((nb_tc, S_BLK, D), lambda i: (0, i, 0)),
            pl.BlockSpec((S_BLK, D), lambda i: (i, 0)),
        ],
        out_specs=pl.BlockSpec((nb_tc, S_BLK, D), lambda i: (0, i, 0)),
        out_shape=jax.ShapeDtypeStruct((nb_tc, S, D), x.dtype),
    )(x, emb_s)


_NW = 32
_CHUNK = 16


def _sc_add_body(S, D, b, x_hbm, emb_hbm, out_hbm, xb, eb):
    wid = lax.axis_index("s") * 2 + lax.axis_index("c")
    rows = S // _NW
    base = wid * rows

    def chunk(k, carry):
        s0 = base + k * _CHUNK
        pltpu.sync_copy(x_hbm.at[b, pl.ds(s0, _CHUNK), :], xb)
        pltpu.sync_copy(emb_hbm.at[pl.ds(s0, _CHUNK), :], eb)
        for r in range(_CHUNK):
            for j in range(D // 16):
                sl = pl.ds(j * 16, 16)
                xb[r, sl] = xb[r, sl] + eb[r, sl]
        pltpu.sync_copy(xb, out_hbm.at[pl.ds(s0, _CHUNK), :])
        return carry

    lax.fori_loop(0, rows // _CHUNK, chunk, 0)


def _sc_add(x, emb_s, b):
    B, S, D = x.shape
    mesh = plsc.VectorSubcoreMesh(core_axis_name="c", subcore_axis_name="s")
    fn = functools.partial(
        pl.kernel,
        out_type=jax.ShapeDtypeStruct((S, D), jnp.float32),
        scratch_types=[
            pltpu.VMEM((_CHUNK, D), jnp.float32),
            pltpu.VMEM((_CHUNK, D), jnp.float32),
        ],
        mesh=mesh,
    )(functools.partial(_sc_add_body, S, D, b))
    return fn(x, emb_s)


_SC_BATCHES = 1


def kernel(x, emb):
    B, S, D = x.shape
    emb_s = jax.lax.slice(emb, (0, 0), (S, D))
    nb_tc = B - _SC_BATCHES
    out_tc = _tc_add(x, emb_s, nb_tc)
    outs_sc = [_sc_add(x, emb_s, b)[None] for b in range(nb_tc, B)]
    return jnp.concatenate([out_tc] + outs_sc, axis=0)


# speedup vs baseline: 1.8495x; 1.8495x over previous
import functools

import jax
import jax.numpy as jnp
from jax import lax
from jax.experimental import pallas as pl
from jax.experimental.pallas import tpu as pltpu
from jax.experimental.pallas import tpu_sc as plsc

_NW = 32
_C = 16


def _sc_body(B, S, D, x_hbm, emb_hbm, out_hbm, xb, eb, sx, se, so):
    wid = lax.axis_index("s") * 2 + lax.axis_index("c")
    rows = S // _NW
    base = wid * rows
    nk = rows // _C
    nsteps = nk * B

    def x_slice(t):
        k = t // B
        b = lax.rem(t, B)
        return x_hbm.at[b, pl.ds(base + k * _C, _C), :]

    def out_slice(t):
        k = t // B
        b = lax.rem(t, B)
        return out_hbm.at[b, pl.ds(base + k * _C, _C), :]

    def emb_slice(k):
        return emb_hbm.at[pl.ds(base + k * _C, _C), :]

    def start_x(t, p):
        pltpu.async_copy(x_slice(t), xb[p], sx[p])

    def wait_x(t, p):
        pltpu.make_async_copy(x_slice(t), xb[p], sx[p]).wait()

    def start_out(t, p):
        pltpu.async_copy(xb[p], out_slice(t), so[p])

    def wait_out(t, p):
        pltpu.make_async_copy(xb[p], out_slice(t), so[p]).wait()

    def start_emb(k, q):
        pltpu.async_copy(emb_slice(k), eb[q], se[q])

    def wait_emb(k, q):
        pltpu.make_async_copy(emb_slice(k), eb[q], se[q]).wait()

    start_emb(0, 0)
    start_emb(1, 1)
    for b in range(B):
        start_x(b, b)

    def group(g, carry):
        for kk in range(2):
            k = 2 * g + kk
            wait_emb(k, kk)
            for b in range(B):
                t = k * B + b
                p = B * kk + b
                pq = B * (1 - kk) + b
                wait_x(t, p)

                def rowblk(r4, c):
                    for dr in range(4):
                        for j in range(D // 16):
                            sl = pl.ds(j * 16, 16)
                            e = eb[kk][r4 * 4 + dr, sl]
                            plsc.addupdate(xb[p].at[r4 * 4 + dr, sl], e)
                    return c

                lax.fori_loop(0, _C // 4, rowblk, 0)
                start_out(t, p)

                @pl.when(t >= B)
                def _():
                    wait_out(t - B, pq)

                @pl.when(t + B < nsteps)
                def _():
                    start_x(t + B, pq)

            @pl.when(k + 2 < nk)
            def _():
                start_emb(k + 2, kk)
        return carry

    lax.fori_loop(0, nk // 2, group, 0)
    for b in range(B):
        t = nsteps - B + b
        wait_out(t, B * ((t // B) % 2) + b)


def kernel(x, emb):
    B, S, D = x.shape
    emb_s = jax.lax.slice(emb, (0, 0), (S, D))
    mesh = plsc.VectorSubcoreMesh(core_axis_name="c", subcore_axis_name="s")

    def body(x_hbm, emb_hbm, out_hbm, *scratch):
        xb = scratch[0:2 * B]
        eb = scratch[2 * B:2 * B + 2]
        sems = scratch[2 * B + 2:]
        sx = sems[0:2 * B]
        so = sems[2 * B:4 * B]
        se = sems[4 * B:4 * B + 2]
        _sc_body(B, S, D, x_hbm, emb_hbm, out_hbm, xb, eb, sx, se, so)

    fn = pl.kernel(
        body,
        out_type=jax.ShapeDtypeStruct((B, S, D), jnp.float32),
        scratch_types=(
            [pltpu.VMEM((_C, D), jnp.float32)] * (2 * B + 2)
            + [pltpu.SemaphoreType.DMA] * (4 * B + 2)
        ),
        mesh=mesh,
    )
    return fn(x, emb_s)


# speedup vs baseline: 2.9890x; 1.6162x over previous
import jax
import jax.numpy as jnp
from jax.experimental import pallas as pl
from jax.experimental.pallas import tpu as pltpu

_S_BLK = 512
_NBUF = 4
_UNROLL = 4


def _body(x_hbm, emb_hbm, out_hbm, xb, eb, sx, se, so):
    B = x_hbm.shape[0]
    S = x_hbm.shape[1]
    nsteps = S // _S_BLK
    g = pl.program_id(0)

    def x_sl(t):
        return x_hbm.at[:, pl.ds(t * _S_BLK, _S_BLK), :]

    def e_sl(t):
        return emb_hbm.at[pl.ds(t * _S_BLK, _S_BLK), :]

    def o_sl(t):
        return out_hbm.at[:, pl.ds(t * _S_BLK, _S_BLK), :]

    def start_in(t, p):
        pltpu.make_async_copy(x_sl(t), xb[p], sx[p]).start()
        pltpu.make_async_copy(e_sl(t), eb[p], se[p]).start()

    def wait_in(t, p):
        pltpu.make_async_copy(x_sl(t), xb[p], sx[p]).wait()
        pltpu.make_async_copy(e_sl(t), eb[p], se[p]).wait()

    def start_out(t, p):
        pltpu.make_async_copy(xb[p], o_sl(t), so[p]).start()

    def wait_out(t, p):
        pltpu.make_async_copy(xb[p], o_sl(t), so[p]).wait()

    for p in range(_UNROLL):
        t = g * _UNROLL + p
        pn = (p + 2) % _NBUF

        @pl.when(t == 0)
        def _():
            start_in(0, 0)
            start_in(1, 1)

        @pl.when(t >= 2)
        def _():
            wait_out(t - 2, pn)

        @pl.when(t + 2 < nsteps)
        def _():
            start_in(t + 2, pn)

        wait_in(t, p)
        xb[p][...] = xb[p][...] + eb[p][...]
        start_out(t, p)

        @pl.when(t == nsteps - 2)
        def _():
            wait_out(t, p)

        @pl.when(t == nsteps - 1)
        def _():
            wait_out(t, p)


def kernel(x, emb):
    B, S, D = x.shape
    emb_s = jax.lax.slice(emb, (0, 0), (S, D))
    nsteps = S // _S_BLK

    def body(x_hbm, emb_hbm, out_hbm, *scratch):
        xb = scratch[0:_NBUF]
        eb = scratch[_NBUF:2 * _NBUF]
        sems = scratch[2 * _NBUF:]
        sx = sems[0:_NBUF]
        se = sems[_NBUF:2 * _NBUF]
        so = sems[2 * _NBUF:3 * _NBUF]
        _body(x_hbm, emb_hbm, out_hbm, xb, eb, sx, se, so)

    hbm = pltpu.MemorySpace.HBM
    return pl.pallas_call(
        body,
        grid=(nsteps // _UNROLL,),
        in_specs=[
            pl.BlockSpec(memory_space=hbm),
            pl.BlockSpec(memory_space=hbm),
        ],
        out_specs=pl.BlockSpec(memory_space=hbm),
        out_shape=jax.ShapeDtypeStruct((B, S, D), x.dtype),
        scratch_shapes=(
            [pltpu.VMEM((B, _S_BLK, D), jnp.float32)] * _NBUF
            + [pltpu.VMEM((_S_BLK, D), jnp.float32)] * _NBUF
            + [pltpu.SemaphoreType.DMA] * (3 * _NBUF)
        ),
    )(x, emb_s)


# speedup vs baseline: 3.0005x; 1.0038x over previous
import jax
import jax.numpy as jnp
from jax.experimental import pallas as pl


def _add_kernel(x_ref, emb_ref, o_ref):
    o_ref[...] = x_ref[...] + emb_ref[...]


def kernel(x, emb):
    B, S, D = x.shape
    S_BLK = 1024
    assert S % S_BLK == 0
    emb_s = jax.lax.slice(emb, (0, 0), (S, D))
    return pl.pallas_call(
        _add_kernel,
        grid=(S // S_BLK,),
        in_specs=[
            pl.BlockSpec((B, S_BLK, D), lambda i: (0, i, 0)),
            pl.BlockSpec((S_BLK, D), lambda i: (i, 0)),
        ],
        out_specs=pl.BlockSpec((B, S_BLK, D), lambda i: (0, i, 0)),
        out_shape=jax.ShapeDtypeStruct((B, S, D), x.dtype),
    )(x, emb_s)
